# Initial kernel scaffold; baseline (speedup 1.0000x reference)
import math

import jax
import jax.numpy as jnp
from jax import lax
from jax.experimental import pallas as pl
from jax.experimental.pallas import tpu as pltpu
from jax.experimental.pallas import tpu_sc as plsc

SOFTMAX_DIM = 32000
PADDING_TOKEN = 0
SMOOTHING_FACTOR = 0.1
CONFIDENCE = 1.0 - SMOOTHING_FACTOR
REDIST = SMOOTHING_FACTOR / (SOFTMAX_DIM - 2)
N_TOKENS = 2048
K_CONST = (SOFTMAX_DIM - 2) * REDIST * math.log(REDIST) + CONFIDENCE * math.log(CONFIDENCE)

LANES = 128
ROWS_FLAT = N_TOKENS * (SOFTMAX_DIM // LANES)
ROW_STRIDE = SOFTMAX_DIM // LANES

NW = 32
TOK_PER_W = N_TOKENS // NW

R_BLK = 128
G_BLK = N_TOKENS // R_BLK


def _sc_gather_kernel(x_hbm, tgt_hbm, out_hbm, tgt_v, idx_v, rows_v, val_v, sem):
    wid = lax.axis_index("s") * 2 + lax.axis_index("c")
    base = wid * TOK_PER_W
    pltpu.sync_copy(tgt_hbm.at[pl.ds(base, TOK_PER_W)], tgt_v)
    for j in range(TOK_PER_W // 16):
        t = tgt_v[pl.ds(j * 16, 16)]
        row_ids = (base + j * 16) + lax.iota(jnp.int32, 16)
        idx_v[pl.ds(j * 16, 16)] = row_ids * ROW_STRIDE + lax.shift_right_logical(t, 7)
    pltpu.async_copy(x_hbm.at[idx_v], rows_v, sem).wait()
    for j in range(TOK_PER_W // 16):
        t = tgt_v[pl.ds(j * 16, 16)]
        lane = lax.bitwise_and(t, 127)
        row_local = (j * 16) + lax.iota(jnp.int32, 16)
        val_v[pl.ds(j * 16, 16)] = plsc.load_gather(rows_v, [row_local, lane])
    pltpu.sync_copy(val_v, out_hbm.at[pl.ds(base, TOK_PER_W)])


def _sc_gather(x_flat, tgt):
    mesh = plsc.VectorSubcoreMesh(core_axis_name="c", subcore_axis_name="s")
    return pl.kernel(
        _sc_gather_kernel,
        mesh=mesh,
        out_type=jax.ShapeDtypeStruct((N_TOKENS,), jnp.float32),
        scratch_types=[
            pltpu.VMEM((TOK_PER_W,), jnp.int32),
            pltpu.VMEM((TOK_PER_W,), jnp.int32),
            pltpu.VMEM((TOK_PER_W, LANES), jnp.float32),
            pltpu.VMEM((TOK_PER_W,), jnp.float32),
            pltpu.SemaphoreType.DMA,
        ],
    )(x_flat, tgt)


def _tc_body(x_ref, tgt_ref, g_ref, out_ref):
    i = pl.program_id(0)
    x = x_ref[...]
    rs = jnp.sum(x, axis=1)
    x0 = x[:, 0]
    t = tgt_ref[0, 0, :]
    gv = g_ref[0, 0, :]
    contrib = jnp.where(
        t != PADDING_TOKEN,
        K_CONST + REDIST * (x0 - rs) - (CONFIDENCE - REDIST) * gv,
        0.0,
    )
    partial = jnp.sum(contrib)

    @pl.when(i == 0)
    def _init():
        out_ref[0, 0] = 0.0

    out_ref[0, 0] += partial


def _tc_reduce(x, tgt3, g3):
    return pl.pallas_call(
        _tc_body,
        grid=(G_BLK,),
        in_specs=[
            pl.BlockSpec((R_BLK, SOFTMAX_DIM), lambda i: (i, 0)),
            pl.BlockSpec((1, 1, R_BLK), lambda i: (i, 0, 0)),
            pl.BlockSpec((1, 1, R_BLK), lambda i: (i, 0, 0)),
        ],
        out_specs=pl.BlockSpec(memory_space=pltpu.SMEM),
        out_shape=jax.ShapeDtypeStruct((1, 1), jnp.float32),
    )(x, tgt3, g3)


def kernel(x, tgt_tokens):
    tgt = tgt_tokens.astype(jnp.int32)
    g = _sc_gather(x.reshape(ROWS_FLAT, LANES), tgt)
    tgt3 = tgt.reshape(G_BLK, 1, R_BLK)
    g3 = g.reshape(G_BLK, 1, R_BLK)
    out = _tc_reduce(x, tgt3, g3)
    return out[0, 0]


# speedup vs baseline: 2.4681x; 2.4681x over previous
import math

import jax
import jax.numpy as jnp
from jax import lax
from jax.experimental import pallas as pl
from jax.experimental.pallas import tpu as pltpu
from jax.experimental.pallas import tpu_sc as plsc

SOFTMAX_DIM = 32000
PADDING_TOKEN = 0
SMOOTHING_FACTOR = 0.1
CONFIDENCE = 1.0 - SMOOTHING_FACTOR
REDIST = SMOOTHING_FACTOR / (SOFTMAX_DIM - 2)
N_TOKENS = 2048
K_CONST = (SOFTMAX_DIM - 2) * REDIST * math.log(REDIST) + CONFIDENCE * math.log(CONFIDENCE)

LANES = 128
ROWS_FLAT = N_TOKENS * (SOFTMAX_DIM // LANES)
ROW_STRIDE = SOFTMAX_DIM // LANES

NW = 32
TOK_PER_W = N_TOKENS // NW

R_BLK = 128
G_BLK = N_TOKENS // R_BLK


def _sc_gather_kernel(x_hbm, tgt_hbm, out_hbm, tgt_v, idx_v, rows_v, val_v, sem):
    wid = lax.axis_index("s") * 2 + lax.axis_index("c")
    base = wid * TOK_PER_W
    pltpu.sync_copy(tgt_hbm.at[pl.ds(base, TOK_PER_W)], tgt_v)
    for j in range(TOK_PER_W // 16):
        t = tgt_v[pl.ds(j * 16, 16)]
        row_ids = (base + j * 16) + lax.iota(jnp.int32, 16)
        idx_v[pl.ds(j * 16, 16)] = row_ids * ROW_STRIDE + lax.shift_right_logical(t, 7)
    pltpu.async_copy(x_hbm.at[idx_v], rows_v, sem).wait()
    for j in range(TOK_PER_W // 16):
        t = tgt_v[pl.ds(j * 16, 16)]
        lane = lax.bitwise_and(t, 127)
        row_local = (j * 16) + lax.iota(jnp.int32, 16)
        val_v[pl.ds(j * 16, 16)] = plsc.load_gather(rows_v, [row_local, lane])
    pltpu.sync_copy(val_v, out_hbm.at[pl.ds(base, TOK_PER_W)])


def _sc_gather(x_flat, tgt):
    mesh = plsc.VectorSubcoreMesh(core_axis_name="c", subcore_axis_name="s")
    return pl.kernel(
        _sc_gather_kernel,
        mesh=mesh,
        compiler_params=pltpu.CompilerParams(needs_layout_passes=False),
        out_type=jax.ShapeDtypeStruct((N_TOKENS,), jnp.float32),
        scratch_types=[
            pltpu.VMEM((TOK_PER_W,), jnp.int32),
            pltpu.VMEM((TOK_PER_W,), jnp.int32),
            pltpu.VMEM((TOK_PER_W, LANES), jnp.float32),
            pltpu.VMEM((TOK_PER_W,), jnp.float32),
            pltpu.SemaphoreType.DMA,
        ],
    )(x_flat, tgt)


def _tc_body(x_ref, tgt_ref, g_ref, out_ref):
    i = pl.program_id(0)
    x = x_ref[...]
    rs = jnp.sum(x, axis=1)
    x0 = x[:, 0]
    t = tgt_ref[0, 0, :]
    gv = g_ref[0, 0, :]
    contrib = jnp.where(
        t != PADDING_TOKEN,
        K_CONST + REDIST * (x0 - rs) - (CONFIDENCE - REDIST) * gv,
        0.0,
    )
    partial = jnp.sum(contrib)

    @pl.when(i == 0)
    def _init():
        out_ref[0, 0] = 0.0

    out_ref[0, 0] += partial


def _tc_reduce(x, tgt3, g3):
    return pl.pallas_call(
        _tc_body,
        grid=(G_BLK,),
        in_specs=[
            pl.BlockSpec((R_BLK, SOFTMAX_DIM), lambda i: (i, 0)),
            pl.BlockSpec((1, 1, R_BLK), lambda i: (i, 0, 0)),
            pl.BlockSpec((1, 1, R_BLK), lambda i: (i, 0, 0)),
        ],
        out_specs=pl.BlockSpec(memory_space=pltpu.SMEM),
        out_shape=jax.ShapeDtypeStruct((1, 1), jnp.float32),
    )(x, tgt3, g3)


def kernel(x, tgt_tokens):
    tgt = tgt_tokens.astype(jnp.int32)
    g = _sc_gather(x.reshape(ROWS_FLAT, LANES), tgt)
    tgt3 = tgt.reshape(G_BLK, 1, R_BLK)
    g3 = g.reshape(G_BLK, 1, R_BLK)
    out = _tc_reduce(x, tgt3, g3)
    return out[0, 0]


# speedup vs baseline: 2.4755x; 1.0030x over previous
import math

import jax
import jax.numpy as jnp
from jax import lax
from jax.experimental import pallas as pl
from jax.experimental.pallas import tpu as pltpu
from jax.experimental.pallas import tpu_sc as plsc

SOFTMAX_DIM = 32000
PADDING_TOKEN = 0
SMOOTHING_FACTOR = 0.1
CONFIDENCE = 1.0 - SMOOTHING_FACTOR
REDIST = SMOOTHING_FACTOR / (SOFTMAX_DIM - 2)
N_TOKENS = 2048
K_CONST = (SOFTMAX_DIM - 2) * REDIST * math.log(REDIST) + CONFIDENCE * math.log(CONFIDENCE)

LANES = 128
ROWS_FLAT = N_TOKENS * (SOFTMAX_DIM // LANES)
ROW_STRIDE = SOFTMAX_DIM // LANES

NW = 32
TOK_PER_W = N_TOKENS // NW

R_BLK = 128
G_BLK = N_TOKENS // R_BLK


def _sc_gather_kernel(x_hbm, tgt_hbm, out_hbm, tgt_v, idx_v, rows_v, val_v, sem):
    wid = lax.axis_index("s") * 2 + lax.axis_index("c")
    base = wid * TOK_PER_W
    pltpu.sync_copy(tgt_hbm.at[pl.ds(base, TOK_PER_W)], tgt_v)
    for j in range(TOK_PER_W // 16):
        t = tgt_v[pl.ds(j * 16, 16)]
        row_ids = (base + j * 16) + lax.iota(jnp.int32, 16)
        idx_v[pl.ds(j * 16, 16)] = row_ids * ROW_STRIDE + lax.shift_right_logical(t, 7)
    pltpu.async_copy(x_hbm.at[idx_v], rows_v, sem).wait()
    for j in range(TOK_PER_W // 16):
        t = tgt_v[pl.ds(j * 16, 16)]
        lane = lax.bitwise_and(t, 127)
        row_local = (j * 16) + lax.iota(jnp.int32, 16)
        val_v[pl.ds(j * 16, 16)] = plsc.load_gather(rows_v, [row_local, lane])
    pltpu.sync_copy(val_v, out_hbm.at[pl.ds(base, TOK_PER_W)])


def _sc_gather(x_flat, tgt):
    mesh = plsc.VectorSubcoreMesh(core_axis_name="c", subcore_axis_name="s")
    return pl.kernel(
        _sc_gather_kernel,
        mesh=mesh,
        compiler_params=pltpu.CompilerParams(needs_layout_passes=False),
        out_type=jax.ShapeDtypeStruct((N_TOKENS,), jnp.float32),
        scratch_types=[
            pltpu.VMEM((TOK_PER_W,), jnp.int32),
            pltpu.VMEM((TOK_PER_W,), jnp.int32),
            pltpu.VMEM((TOK_PER_W, LANES), jnp.float32),
            pltpu.VMEM((TOK_PER_W,), jnp.float32),
            pltpu.SemaphoreType.DMA,
        ],
    )(x_flat, tgt)


N_PANELS = 5
PANEL_W = SOFTMAX_DIM // N_PANELS


def _tc_body(*refs):
    panel_refs = refs[:N_PANELS]
    tgt_ref, g_ref, out_ref = refs[N_PANELS:]
    i = pl.program_id(0)
    rs = panel_refs[0][...].sum(axis=1)
    for p in panel_refs[1:]:
        rs = rs + p[...].sum(axis=1)
    x0 = panel_refs[0][:, 0]
    t = tgt_ref[0, 0, :]
    gv = g_ref[0, 0, :]
    contrib = jnp.where(
        t != PADDING_TOKEN,
        K_CONST + REDIST * (x0 - rs) - (CONFIDENCE - REDIST) * gv,
        0.0,
    )
    partial = jnp.sum(contrib)

    @pl.when(i == 0)
    def _init():
        out_ref[0, 0] = 0.0

    out_ref[0, 0] += partial


def _tc_reduce(x, tgt3, g3):
    panel_specs = [
        pl.BlockSpec((R_BLK, PANEL_W), lambda i, k=k: (i, k))
        for k in range(N_PANELS)
    ]
    return pl.pallas_call(
        _tc_body,
        grid=(G_BLK,),
        in_specs=panel_specs + [
            pl.BlockSpec((1, 1, R_BLK), lambda i: (i, 0, 0)),
            pl.BlockSpec((1, 1, R_BLK), lambda i: (i, 0, 0)),
        ],
        out_specs=pl.BlockSpec(memory_space=pltpu.SMEM),
        out_shape=jax.ShapeDtypeStruct((1, 1), jnp.float32),
    )(*([x] * N_PANELS), tgt3, g3)


def kernel(x, tgt_tokens):
    tgt = tgt_tokens.astype(jnp.int32)
    g = _sc_gather(x.reshape(ROWS_FLAT, LANES), tgt)
    tgt3 = tgt.reshape(G_BLK, 1, R_BLK)
    g3 = g.reshape(G_BLK, 1, R_BLK)
    out = _tc_reduce(x, tgt3, g3)
    return out[0, 0]
